# Initial kernel scaffold; baseline (speedup 1.0000x reference)
import jax
import jax.numpy as jnp
from jax.experimental import pallas as pl


def kernel(x, W_topk, b_topk, W_noisy, b_noisy):
    raise NotImplementedError("write your pallas kernel here")


# speedup vs baseline: 4.9151x; 4.9151x over previous
import functools

import jax
import jax.numpy as jnp
from jax.experimental import pallas as pl
from jax.experimental.pallas import tpu as pltpu

D_MODEL = 4096
EXPERTS = 64
TOPK = 8
BLK = 512


def _router_tc_kernel(x_ref, w_ref, b_ref, probs_ref, idx_ref):
    x = x_ref[...]
    w = w_ref[...]
    b = b_ref[...]
    logits = jnp.dot(x, w, preferred_element_type=jnp.float32) + b

    r = logits.shape[0]
    col = jax.lax.broadcasted_iota(jnp.int32, (r, EXPERTS), 1)
    neg_inf = jnp.float32(-jnp.inf)

    cur = logits
    sel = jnp.zeros((r, EXPERTS), jnp.bool_)
    idx_cols = []
    for _ in range(TOPK):
        m = jnp.max(cur, axis=-1, keepdims=True)
        hit = cur == m
        idx = jnp.min(jnp.where(hit, col, EXPERTS), axis=-1, keepdims=True)
        idx_cols.append(idx)
        chosen = col == idx
        sel = sel | chosen
        cur = jnp.where(chosen, neg_inf, cur)

    m0 = jnp.max(logits, axis=-1, keepdims=True)
    e = jnp.exp(logits - m0)
    z = jnp.sum(jnp.where(sel, e, 0.0), axis=-1, keepdims=True)
    probs_ref[...] = jnp.where(sel, e / z, 0.0)
    idx_ref[...] = jnp.concatenate(idx_cols, axis=1)


@jax.jit
def kernel(x, W_topk, b_topk, W_noisy, b_noisy):
    del W_noisy, b_noisy
    B, S, D = x.shape
    rows = B * S
    x2 = x.reshape(rows, D)
    b2 = b_topk.reshape(1, EXPERTS)

    grid = (rows // BLK,)
    probs, idx = pl.pallas_call(
        _router_tc_kernel,
        grid=grid,
        in_specs=[
            pl.BlockSpec((BLK, D), lambda i: (i, 0)),
            pl.BlockSpec((D, EXPERTS), lambda i: (0, 0)),
            pl.BlockSpec((1, EXPERTS), lambda i: (0, 0)),
        ],
        out_specs=[
            pl.BlockSpec((BLK, EXPERTS), lambda i: (i, 0)),
            pl.BlockSpec((BLK, TOPK), lambda i: (i, 0)),
        ],
        out_shape=[
            jax.ShapeDtypeStruct((rows, EXPERTS), jnp.float32),
            jax.ShapeDtypeStruct((rows, TOPK), jnp.int32),
        ],
        compiler_params=pltpu.CompilerParams(
            dimension_semantics=("arbitrary",),
        ),
    )(x2, W_topk, b2)

    return probs.reshape(B, S, EXPERTS), idx.reshape(B, S, TOPK)


# speedup vs baseline: 5.3143x; 1.0812x over previous
import functools

import jax
import jax.numpy as jnp
from jax.experimental import pallas as pl
from jax.experimental.pallas import tpu as pltpu

D_MODEL = 4096
EXPERTS = 64
TOPK = 8
BLK = 512


def _router_tc_kernel(x_ref, w_ref, b_ref, probs_ref, idx_ref):
    x = x_ref[...]
    w = w_ref[...]
    b = b_ref[...]
    logits = jnp.dot(x, w, preferred_element_type=jnp.float32) + b

    r = logits.shape[0]
    col = jax.lax.broadcasted_iota(jnp.int32, (r, EXPERTS), 1)

    bits = jax.lax.bitcast_convert_type(logits, jnp.int32)
    key = jnp.where(bits < 0, bits ^ jnp.int32(0x7FFFFFFF), bits)
    key = (key & jnp.int32(~63)) | (jnp.int32(EXPERTS - 1) - col)

    cur = key
    sel = jnp.zeros((r, EXPERTS), jnp.bool_)
    idx_cols = []
    for _ in range(TOPK):
        m = jnp.max(cur, axis=-1, keepdims=True)
        idx_cols.append(jnp.int32(EXPERTS - 1) - (m & jnp.int32(63)))
        chosen = cur == m
        sel = sel | chosen
        cur = jnp.where(chosen, jnp.int32(-(2**31)), cur)

    m0 = jnp.max(logits, axis=-1, keepdims=True)
    e = jnp.exp(logits - m0)
    z = jnp.sum(jnp.where(sel, e, 0.0), axis=-1, keepdims=True)
    probs_ref[...] = jnp.where(sel, e / z, 0.0)
    idx_ref[...] = jnp.concatenate(idx_cols, axis=1)


@jax.jit
def kernel(x, W_topk, b_topk, W_noisy, b_noisy):
    del W_noisy, b_noisy
    B, S, D = x.shape
    rows = B * S
    x2 = x.reshape(rows, D)
    b2 = b_topk.reshape(1, EXPERTS)

    grid = (rows // BLK,)
    probs, idx = pl.pallas_call(
        _router_tc_kernel,
        grid=grid,
        in_specs=[
            pl.BlockSpec((BLK, D), lambda i: (i, 0)),
            pl.BlockSpec((D, EXPERTS), lambda i: (0, 0)),
            pl.BlockSpec((1, EXPERTS), lambda i: (0, 0)),
        ],
        out_specs=[
            pl.BlockSpec((BLK, EXPERTS), lambda i: (i, 0)),
            pl.BlockSpec((BLK, TOPK), lambda i: (i, 0)),
        ],
        out_shape=[
            jax.ShapeDtypeStruct((rows, EXPERTS), jnp.float32),
            jax.ShapeDtypeStruct((rows, TOPK), jnp.int32),
        ],
        compiler_params=pltpu.CompilerParams(
            dimension_semantics=("arbitrary",),
        ),
    )(x2, W_topk, b2)

    return probs.reshape(B, S, EXPERTS), idx.reshape(B, S, TOPK)


# speedup vs baseline: 5.8042x; 1.0922x over previous
import functools

import jax
import jax.numpy as jnp
from jax.experimental import pallas as pl
from jax.experimental.pallas import tpu as pltpu

D_MODEL = 4096
EXPERTS = 64
TOPK = 8
BLK = 1024


def _router_tc_kernel(x_ref, w_ref, b_ref, probs_ref, idx_ref):
    x = x_ref[...]
    w = w_ref[...]
    b = b_ref[...]
    logits = jnp.dot(x, w, preferred_element_type=jnp.float32) + b

    r = logits.shape[0]
    col = jax.lax.broadcasted_iota(jnp.int32, (r, EXPERTS), 1)

    bits = jax.lax.bitcast_convert_type(logits, jnp.int32)
    key = jnp.where(bits < 0, bits ^ jnp.int32(0x7FFFFFFF), bits)
    key = (key & jnp.int32(~63)) | (jnp.int32(EXPERTS - 1) - col)

    cur = key
    sel = jnp.zeros((r, EXPERTS), jnp.bool_)
    idx_cols = []
    for _ in range(TOPK):
        m = jnp.max(cur, axis=-1, keepdims=True)
        idx_cols.append(jnp.int32(EXPERTS - 1) - (m & jnp.int32(63)))
        chosen = cur == m
        sel = sel | chosen
        cur = jnp.where(chosen, jnp.int32(-(2**31)), cur)

    m0 = jnp.max(logits, axis=-1, keepdims=True)
    e = jnp.exp(logits - m0)
    z = jnp.sum(jnp.where(sel, e, 0.0), axis=-1, keepdims=True)
    probs_ref[...] = jnp.where(sel, e / z, 0.0)
    idx_ref[...] = jnp.concatenate(idx_cols, axis=1)


@jax.jit
def kernel(x, W_topk, b_topk, W_noisy, b_noisy):
    del W_noisy, b_noisy
    B, S, D = x.shape
    rows = B * S
    x2 = x.reshape(rows, D)
    b2 = b_topk.reshape(1, EXPERTS)

    grid = (rows // BLK,)
    probs, idx = pl.pallas_call(
        _router_tc_kernel,
        grid=grid,
        in_specs=[
            pl.BlockSpec((BLK, D), lambda i: (i, 0)),
            pl.BlockSpec((D, EXPERTS), lambda i: (0, 0)),
            pl.BlockSpec((1, EXPERTS), lambda i: (0, 0)),
        ],
        out_specs=[
            pl.BlockSpec((BLK, EXPERTS), lambda i: (i, 0)),
            pl.BlockSpec((BLK, TOPK), lambda i: (i, 0)),
        ],
        out_shape=[
            jax.ShapeDtypeStruct((rows, EXPERTS), jnp.float32),
            jax.ShapeDtypeStruct((rows, TOPK), jnp.int32),
        ],
        compiler_params=pltpu.CompilerParams(
            dimension_semantics=("arbitrary",),
        ),
    )(x2, W_topk, b2)

    return probs.reshape(B, S, EXPERTS), idx.reshape(B, S, TOPK)
